# speedup vs baseline: 31.1867x; 31.1867x over previous
import jax
import jax.numpy as jnp
from jax import lax
from jax.experimental import pallas as pl

B, P, C, O = 32, 8732, 21, 20
PPAD = 9216
TROWS = 24
THRESH = 0.5


def _match_encode_body(p4_ref, ttm_ref, tcm_ref, loc_ref, conf_ref):
    p4 = p4_ref[...]
    pcx, pcy = p4[0:1, :], p4[1:2, :]
    pw, ph = p4[2:3, :], p4[3:4, :]
    px1 = pcx - pw / 2.0
    py1 = pcy - ph / 2.0
    px2 = pcx + pw / 2.0
    py2 = pcy + ph / 2.0

    ttm = ttm_ref[0]
    tx1 = ttm[:, 0:1]
    ty1 = ttm[:, 1:2]
    tx2 = ttm[:, 2:3]
    ty2 = ttm[:, 3:4]

    iw = jnp.clip(jnp.minimum(tx2, px2) - jnp.maximum(tx1, px1), 0.0, None)
    ih = jnp.clip(jnp.minimum(ty2, py2) - jnp.maximum(ty1, py1), 0.0, None)
    inter = iw * ih
    area_a = (tx2 - tx1) * (ty2 - ty1)
    area_b = (px2 - px1) * (py2 - py1)
    union = area_a + area_b - inter
    ov = inter / union

    rows = lax.broadcasted_iota(jnp.int32, (TROWS, PPAD), 0)
    lanes = lax.broadcasted_iota(jnp.int32, (TROWS, PPAD), 1)

    bto = jnp.max(ov, axis=0, keepdims=True)
    bti = jnp.min(jnp.where(ov == bto, rows, TROWS),
                  axis=0, keepdims=True)
    bpv = jnp.max(ov, axis=1, keepdims=True)
    bpi = jnp.min(jnp.where(ov == bpv, lanes, PPAD),
                  axis=1, keepdims=True)

    match = (lanes == bpi) & (rows < O)
    j_forced = jnp.max(jnp.where(match, rows, -1), axis=0, keepdims=True)
    bti2 = jnp.where(j_forced >= 0, j_forced, bti)
    maskok = (bto >= THRESH) | (j_forced >= 0)

    rows128 = lax.broadcasted_iota(jnp.int32, (128, PPAD), 0)
    onehot = (rows128 == bti2).astype(jnp.float32)
    tcm = tcm_ref[0]
    sx = tcm[0:1, :] + tcm[2:3, :]
    sy = tcm[1:2, :] + tcm[3:4, :]
    dx = tcm[2:3, :] - tcm[0:1, :]
    dy = tcm[3:4, :] - tcm[1:2, :]
    lb = tcm[4:5, :]
    tm = jnp.concatenate([sx, sy, dx, dy, lb, jnp.zeros((3, 128), jnp.float32)])
    m = lax.dot_general(tm, onehot, (((1,), (0,)), ((), ())),
                        precision=lax.Precision.HIGHEST)

    m_s = m[0:2, :]
    m_d = m[2:4, :]
    pcxy = p4[0:2, :]
    pwh = p4[2:4, :]
    g_c = (m_s / 2.0 - pcxy) / (0.1 * pwh)
    g_wh = jnp.log(m_d / pwh) / 0.2
    loc_ref[0] = jnp.concatenate([g_c, g_wh])
    conf = jnp.where(maskok, m[4:5, :] + 1.0, 0.0).astype(jnp.int32)
    conf_ref[0] = conf


def kernel(loc_data, conf_data, priors, targets):
    del loc_data, conf_data
    pri = priors[:P, :]
    pad_pri = jnp.broadcast_to(jnp.array([-50.0, -50.0, 1.0, 1.0],
                                         jnp.float32), (PPAD - P, 4))
    p4 = jnp.concatenate([pri, pad_pri], axis=0).T

    pad_box = jnp.array([-9.0, -9.0, -8.0, -8.0, 0.0], jnp.float32)
    tgt = jnp.concatenate(
        [targets, jnp.broadcast_to(pad_box, (B, TROWS - O, 5))], axis=1)
    ttm = jnp.pad(tgt, ((0, 0), (0, 0), (0, 128 - 5)))
    tcm = jnp.pad(jnp.swapaxes(tgt, 1, 2),
                  ((0, 0), (0, 8 - 5), (0, 128 - TROWS)))

    loc_p, conf_p = pl.pallas_call(
        _match_encode_body,
        grid=(B,),
        in_specs=[
            pl.BlockSpec((4, PPAD), lambda i: (0, 0)),
            pl.BlockSpec((1, TROWS, 128), lambda i: (i, 0, 0)),
            pl.BlockSpec((1, 8, 128), lambda i: (i, 0, 0)),
        ],
        out_specs=[
            pl.BlockSpec((1, 4, PPAD), lambda i: (i, 0, 0)),
            pl.BlockSpec((1, 1, PPAD), lambda i: (i, 0, 0)),
        ],
        out_shape=[
            jax.ShapeDtypeStruct((B, 4, PPAD), jnp.float32),
            jax.ShapeDtypeStruct((B, 1, PPAD), jnp.int32),
        ],
    )(p4, ttm, tcm)

    loc_t = jnp.swapaxes(loc_p, 1, 2)[:, :P, :]
    conf_t = conf_p[:, 0, :P]
    return loc_t, conf_t


# speedup vs baseline: 32.3372x; 1.0369x over previous
import functools

import jax
import jax.numpy as jnp
from jax import lax
from jax.experimental import pallas as pl
from jax.experimental.pallas import tpu as pltpu
from jax.experimental.pallas import tpu_sc as plsc

B, P, C, O = 32, 8732, 21, 20
PPAD = 9216
TROWS = 24
THRESH = 0.5
NW = 32
NU = PPAD // 128
NS_MAX = 3


def _match_body(p4_ref, ttm_ref, bti_ref, msk_ref):
    p4 = p4_ref[...]
    pcx, pcy = p4[0:1, :], p4[1:2, :]
    pw, ph = p4[2:3, :], p4[3:4, :]
    px1 = pcx - pw / 2.0
    py1 = pcy - ph / 2.0
    px2 = pcx + pw / 2.0
    py2 = pcy + ph / 2.0

    ttm = ttm_ref[0]
    tx1 = ttm[:, 0:1]
    ty1 = ttm[:, 1:2]
    tx2 = ttm[:, 2:3]
    ty2 = ttm[:, 3:4]

    iw = jnp.clip(jnp.minimum(tx2, px2) - jnp.maximum(tx1, px1), 0.0, None)
    ih = jnp.clip(jnp.minimum(ty2, py2) - jnp.maximum(ty1, py1), 0.0, None)
    inter = iw * ih
    area_a = (tx2 - tx1) * (ty2 - ty1)
    area_b = (px2 - px1) * (py2 - py1)
    union = area_a + area_b - inter
    ov = inter / union

    rows = lax.broadcasted_iota(jnp.int32, (TROWS, PPAD), 0)
    lanes = lax.broadcasted_iota(jnp.int32, (TROWS, PPAD), 1)

    bto = jnp.max(ov, axis=0, keepdims=True)
    bti = jnp.min(jnp.where(ov == bto, rows, TROWS),
                  axis=0, keepdims=True)
    bpv = jnp.max(ov, axis=1, keepdims=True)
    bpi = jnp.min(jnp.where(ov == bpv, lanes, PPAD),
                  axis=1, keepdims=True)

    match = (lanes == bpi) & (rows < O)
    j_forced = jnp.max(jnp.where(match, rows, -1), axis=0, keepdims=True)
    bti2 = jnp.where(j_forced >= 0, j_forced, bti)
    maskok = (bto >= THRESH) | (j_forced >= 0)
    bti_ref[0] = bti2
    msk_ref[0] = maskok.astype(jnp.int32)


def _logf16(x):
    bits = lax.bitcast_convert_type(x, jnp.int32)
    e = ((bits >> 23) & 0xFF) - 126
    m = lax.bitcast_convert_type((bits & 0x007FFFFF) | 0x3F000000,
                                 jnp.float32)
    c = m < 0.7071067811865476
    m = jnp.where(c, m + m, m)
    ef = (e - jnp.where(c, 1, 0)).astype(jnp.float32)
    z = m - 1.0
    zz = z * z
    poly = jnp.full((16,), 7.0376836292e-2, jnp.float32)
    for k in (-1.1514610310e-1, 1.1676998740e-1, -1.2420140846e-1,
              1.4249322787e-1, -1.6668057665e-1, 2.0000714765e-1,
              -2.4999993993e-1, 3.3333331174e-1):
        poly = poly * z + jnp.float32(k)
    y = z * zz * poly
    y = y + ef * jnp.float32(-2.12194440e-4)
    y = y - 0.5 * zz
    return z + y + ef * jnp.float32(0.693359375)


def _sc_encode_body(p4_hbm, t_hbm, bti_hbm, msk_hbm, loc_hbm, conf_hbm,
                    pri_v, t_v, bti_v, msk_v, loc_v, conf_v):
    wid = lax.axis_index("s") * 2 + lax.axis_index("c")
    pltpu.sync_copy(t_hbm, t_v)
    lane = lax.broadcasted_iota(jnp.int32, (16,), 0)

    for s in range(NS_MAX):
        unit = s * NW + wid

        def do_unit(unit=unit):
            base = pl.multiple_of(unit * 128, 128)
            pltpu.sync_copy(p4_hbm.at[:, pl.ds(base, 128)], pri_v)
            pltpu.sync_copy(bti_hbm.at[:, pl.ds(base, 128)], bti_v)
            pltpu.sync_copy(msk_hbm.at[:, pl.ds(base, 128)], msk_v)

            def one_image(i, carry):
                tbase = i * 256
                for ch in range(8):
                    cs = ch * 16
                    idx = bti_v[i, pl.ds(cs, 16)]
                    msk = msk_v[i, pl.ds(cs, 16)]
                    ti = tbase + idx
                    x1 = plsc.load_gather(t_v, [ti])
                    y1 = plsc.load_gather(t_v, [ti + 32])
                    x2 = plsc.load_gather(t_v, [ti + 64])
                    y2 = plsc.load_gather(t_v, [ti + 96])
                    lb = plsc.load_gather(t_v, [ti + 128])
                    pcx = pri_v[0, pl.ds(cs, 16)]
                    pcy = pri_v[1, pl.ds(cs, 16)]
                    pw = pri_v[2, pl.ds(cs, 16)]
                    ph = pri_v[3, pl.ds(cs, 16)]
                    g_cx = ((x1 + x2) / 2.0 - pcx) / (0.1 * pw)
                    g_cy = ((y1 + y2) / 2.0 - pcy) / (0.1 * ph)
                    g_w = _logf16((x2 - x1) / pw) / 0.2
                    g_h = _logf16((y2 - y1) / ph) / 0.2
                    conf = jnp.where(msk != 0, (lb + 1.0).astype(jnp.int32), 0)
                    loc_v[i, 0, pl.ds(cs, 16)] = g_cx
                    loc_v[i, 1, pl.ds(cs, 16)] = g_cy
                    loc_v[i, 2, pl.ds(cs, 16)] = g_w
                    loc_v[i, 3, pl.ds(cs, 16)] = g_h
                    conf_v[i, pl.ds(cs, 16)] = conf
                return carry

            lax.fori_loop(0, B, one_image, 0)
            pltpu.sync_copy(loc_v, loc_hbm.at[:, :, pl.ds(base, 128)])
            pltpu.sync_copy(conf_v, conf_hbm.at[:, pl.ds(base, 128)])

        if s < 2:
            do_unit()
        else:
            pl.when(unit < NU)(do_unit)


_SC_ENCODE_CACHE = []


def _sc_encode(*args):
    if not _SC_ENCODE_CACHE:
        _SC_ENCODE_CACHE.append(_make_sc_encode())
    return _SC_ENCODE_CACHE[0](*args)


def _make_sc_encode():
    return functools.partial(
        pl.kernel,
        out_type=[
            jax.ShapeDtypeStruct((B, 4, PPAD), jnp.float32),
            jax.ShapeDtypeStruct((B, PPAD), jnp.int32),
        ],
        mesh=plsc.VectorSubcoreMesh(core_axis_name="c", subcore_axis_name="s",
                                    num_cores=2, num_subcores=16),
        compiler_params=pltpu.CompilerParams(needs_layout_passes=False),
        scratch_types=[
            pltpu.VMEM((4, 128), jnp.float32),
            pltpu.VMEM((B * 8 * 32,), jnp.float32),
            pltpu.VMEM((B, 128), jnp.int32),
            pltpu.VMEM((B, 128), jnp.int32),
            pltpu.VMEM((B, 4, 128), jnp.float32),
            pltpu.VMEM((B, 128), jnp.int32),
        ],
    )(_sc_encode_body)


def kernel(loc_data, conf_data, priors, targets):
    del loc_data, conf_data
    pri = priors[:P, :]
    pad_pri = jnp.broadcast_to(jnp.array([-50.0, -50.0, 1.0, 1.0],
                                         jnp.float32), (PPAD - P, 4))
    p4 = jnp.concatenate([pri, pad_pri], axis=0).T

    pad_box = jnp.array([-9.0, -9.0, -8.0, -8.0, 0.0], jnp.float32)
    tgt24 = jnp.concatenate(
        [targets, jnp.broadcast_to(pad_box, (B, TROWS - O, 5))], axis=1)
    ttm = jnp.pad(tgt24, ((0, 0), (0, 0), (0, 128 - 5)))
    tgt32 = jnp.concatenate(
        [targets, jnp.broadcast_to(pad_box, (B, 32 - O, 5))], axis=1)
    t_sc = jnp.pad(jnp.swapaxes(tgt32, 1, 2),
                   ((0, 0), (0, 3), (0, 0))).reshape(-1)

    bti, msk = pl.pallas_call(
        _match_body,
        grid=(B,),
        in_specs=[
            pl.BlockSpec((4, PPAD), lambda i: (0, 0)),
            pl.BlockSpec((1, TROWS, 128), lambda i: (i, 0, 0)),
        ],
        out_specs=[
            pl.BlockSpec((1, 1, PPAD), lambda i: (i, 0, 0)),
            pl.BlockSpec((1, 1, PPAD), lambda i: (i, 0, 0)),
        ],
        out_shape=[
            jax.ShapeDtypeStruct((B, 1, PPAD), jnp.int32),
            jax.ShapeDtypeStruct((B, 1, PPAD), jnp.int32),
        ],
    )(p4, ttm)

    loc_p, conf_p = _sc_encode(p4, t_sc, bti[:, 0, :], msk[:, 0, :])
    return jnp.swapaxes(loc_p, 1, 2)[:, :P, :], conf_p[:, :P]
